# Initial kernel scaffold; baseline (speedup 1.0000x reference)
import jax
import jax.numpy as jnp
from jax.experimental import pallas as pl


def kernel(box_predictions, class_predictions, anchors):
    raise NotImplementedError("write your pallas kernel here")


# speedup vs baseline: 4.8616x; 4.8616x over previous
import jax
import jax.numpy as jnp
from jax.experimental import pallas as pl
from jax.experimental.pallas import tpu as pltpu

_N = 20000
_C = 80
_B = 2
_K = 1000
_P = 100
_D = 100
_IOU = 0.5
_TH = 0.05
_INV_WH = 1.0 / 512.0
_PL = 128


def _body(cls_ref, bp_ref, anc_ref,
          sco, x1o, y1o, x2o, y2o, clso, vo,
          s_scr, x1s, y1s, x2s, y2s, a2s):
    NEG = jnp.float32(-jnp.inf)
    iota = jax.lax.broadcasted_iota(jnp.int32, (1, _N), 1)
    liota = jax.lax.broadcasted_iota(jnp.int32, (1, _PL), 1)

    anc = anc_ref[...]
    acx, acy, aw, ah = anc[0:1], anc[1:2], anc[2:3], anc[3:4]
    for b in range(_B):
        bp = bp_ref[b]
        cx = bp[0:1] * jnp.float32(0.1) * aw + acx
        cy = bp[1:2] * jnp.float32(0.1) * ah + acy
        w = jnp.exp(bp[2:3] * jnp.float32(0.2)) * aw
        h = jnp.exp(bp[3:4] * jnp.float32(0.2)) * ah
        x1 = (cx - w * 0.5) * _INV_WH
        y1 = (cy - h * 0.5) * _INV_WH
        x2 = (cx + w * 0.5) * _INV_WH
        y2 = (cy + h * 0.5) * _INV_WH
        x1s[b:b + 1, :] = x1
        y1s[b:b + 1, :] = y1
        x2s[b:b + 1, :] = x2
        y2s[b:b + 1, :] = y2
        a2s[b:b + 1, :] = (x2 - x1) * (y2 - y1)

    probs = jax.nn.sigmoid(cls_ref[...])
    bits = jax.lax.bitcast_convert_type(probs, jnp.int32)

    def bs_val(_, lohi):
        lo, hi = lohi
        mid = lo + (hi - lo + 1) // 2
        cnt = jnp.sum((bits >= mid).astype(jnp.int32), axis=1, keepdims=True)
        p = cnt >= _K
        return jnp.where(p, mid, lo), jnp.where(p, hi, mid - 1)

    lo0 = jnp.zeros((_B * _C, 1), jnp.int32)
    hi0 = jnp.full((_B * _C, 1), 0x7F800000, jnp.int32)
    kth, _ = jax.lax.fori_loop(0, 31, bs_val, (lo0, hi0))

    gt = bits > kth
    n_gt = jnp.sum(gt.astype(jnp.int32), axis=1, keepdims=True)
    need = _K - n_gt
    eq = bits == kth

    def bs_idx(_, lohi):
        lo, hi = lohi
        mid = lo + (hi - lo + 1) // 2
        cnt = jnp.sum((eq & (iota < mid)).astype(jnp.int32),
                      axis=1, keepdims=True)
        p = cnt <= need
        return jnp.where(p, mid, lo), jnp.where(p, hi, mid - 1)

    tlo0 = jnp.zeros((_B * _C, 1), jnp.int32)
    thi0 = jnp.full((_B * _C, 1), _N, jnp.int32)
    tcut, _ = jax.lax.fori_loop(0, 15, bs_idx, (tlo0, thi0))

    keep = gt | (eq & (iota < tcut))
    s_scr[...] = jnp.where(keep, probs, NEG)

    BIG = jnp.int32(1 << 30)
    ciota = jax.lax.broadcasted_iota(jnp.int32, (_C, _PL), 0)
    piota = jax.lax.broadcasted_iota(jnp.int32, (_C, _PL), 1)
    fidx = jnp.where(piota < _P, ciota * _P + piota, BIG)

    for b in range(_B):
        r0, r1 = b * _C, (b + 1) * _C
        x1b = x1s[b:b + 1, :]
        y1b = y1s[b:b + 1, :]
        x2b = x2s[b:b + 1, :]
        y2b = y2s[b:b + 1, :]
        a2 = a2s[b:b + 1, :]

        def nms_step(t, sel):
            ssc, sx1, sy1, sx2, sy2 = sel
            s = s_scr[r0:r1, :]
            m = jnp.max(s, axis=1, keepdims=True)
            idx = jnp.min(jnp.where(s == m, iota, _N), axis=1, keepdims=True)
            oh = iota == idx
            bx1 = jnp.sum(jnp.where(oh, x1b, 0.0), axis=1, keepdims=True)
            by1 = jnp.sum(jnp.where(oh, y1b, 0.0), axis=1, keepdims=True)
            bx2 = jnp.sum(jnp.where(oh, x2b, 0.0), axis=1, keepdims=True)
            by2 = jnp.sum(jnp.where(oh, y2b, 0.0), axis=1, keepdims=True)
            wx = jnp.maximum(jnp.minimum(bx2, x2b) - jnp.maximum(bx1, x1b), 0.0)
            wy = jnp.maximum(jnp.minimum(by2, y2b) - jnp.maximum(by1, y1b), 0.0)
            inter = wx * wy
            a1 = (bx2 - bx1) * (by2 - by1)
            iou = inter / (a1 + a2 - inter + jnp.float32(1e-8))
            sup = (iou > _IOU) | oh
            s_scr[r0:r1, :] = jnp.where(sup, NEG, s)
            slot = liota == t
            return (jnp.where(slot, m, ssc),
                    jnp.where(slot, bx1, sx1),
                    jnp.where(slot, by1, sy1),
                    jnp.where(slot, bx2, sx2),
                    jnp.where(slot, by2, sy2))

        sel0 = (jnp.full((_C, _PL), NEG),
                jnp.zeros((_C, _PL), jnp.float32),
                jnp.zeros((_C, _PL), jnp.float32),
                jnp.zeros((_C, _PL), jnp.float32),
                jnp.zeros((_C, _PL), jnp.float32))
        ssc, sx1, sy1, sx2, sy2 = jax.lax.fori_loop(0, _P, nms_step, sel0)

        def fin_step(j, carry):
            fs, vc, osc, ox1, oy1, ox2, oy2, ocl = carry
            m = jnp.max(jnp.max(fs, axis=1, keepdims=True),
                        axis=0, keepdims=True)
            cand = jnp.where(fs == m, fidx, BIG)
            mi = jnp.min(jnp.min(cand, axis=1, keepdims=True),
                         axis=0, keepdims=True)
            oh = fidx == mi
            ex = lambda a: jnp.sum(jnp.sum(jnp.where(oh, a, 0.0),
                                           axis=1, keepdims=True),
                                   axis=0, keepdims=True)
            cid = (mi // _P).astype(jnp.float32)
            ok = m > _TH
            slot = liota == j
            osc = jnp.where(slot, jnp.where(ok, m, 0.0), osc)
            ox1 = jnp.where(slot, jnp.where(ok, ex(sx1), 0.0), ox1)
            oy1 = jnp.where(slot, jnp.where(ok, ex(sy1), 0.0), oy1)
            ox2 = jnp.where(slot, jnp.where(ok, ex(sx2), 0.0), ox2)
            oy2 = jnp.where(slot, jnp.where(ok, ex(sy2), 0.0), oy2)
            ocl = jnp.where(slot, jnp.where(ok, cid, 0.0), ocl)
            vc = vc + ok.astype(jnp.int32)
            fs = jnp.where(oh, NEG, fs)
            return fs, vc, osc, ox1, oy1, ox2, oy2, ocl

        z = jnp.zeros((1, _PL), jnp.float32)
        init = (ssc, jnp.zeros((1, 1), jnp.int32), z, z, z, z, z, z)
        (_, vc, osc, ox1, oy1, ox2, oy2, ocl) = jax.lax.fori_loop(
            0, _D, fin_step, init)

        sco[b:b + 1, :] = osc[:, 0:_D]
        x1o[b:b + 1, :] = ox1[:, 0:_D]
        y1o[b:b + 1, :] = oy1[:, 0:_D]
        x2o[b:b + 1, :] = ox2[:, 0:_D]
        y2o[b:b + 1, :] = oy2[:, 0:_D]
        clso[b:b + 1, :] = ocl[:, 0:_D]
        vo[b:b + 1, :] = vc


def kernel(box_predictions, class_predictions, anchors):
    B, N, _ = box_predictions.shape
    C = class_predictions.shape[-1]
    cls_t = jnp.transpose(class_predictions.astype(jnp.float32),
                          (0, 2, 1)).reshape(B * C, N)
    bp_t = jnp.transpose(box_predictions.astype(jnp.float32), (0, 2, 1))
    anc_t = jnp.transpose(anchors.astype(jnp.float32), (1, 0))

    f32 = jnp.float32
    outs = pl.pallas_call(
        _body,
        out_shape=[
            jax.ShapeDtypeStruct((B, _D), f32),
            jax.ShapeDtypeStruct((B, _D), f32),
            jax.ShapeDtypeStruct((B, _D), f32),
            jax.ShapeDtypeStruct((B, _D), f32),
            jax.ShapeDtypeStruct((B, _D), f32),
            jax.ShapeDtypeStruct((B, _D), f32),
            jax.ShapeDtypeStruct((B, 1), jnp.int32),
        ],
        scratch_shapes=[
            pltpu.VMEM((B * C, N), f32),
            pltpu.VMEM((B, N), f32),
            pltpu.VMEM((B, N), f32),
            pltpu.VMEM((B, N), f32),
            pltpu.VMEM((B, N), f32),
            pltpu.VMEM((B, N), f32),
        ],
    )(cls_t, bp_t, anc_t)

    sc, x1, y1, x2, y2, cls_o, valid = outs
    boxes = jnp.stack([x1, y1, x2, y2], axis=-1)
    return boxes, sc, cls_o, valid[:, 0]
